# baseline (speedup 1.0000x reference)
import functools

import jax
import jax.numpy as jnp
from jax import lax
from jax.experimental import pallas as pl
from jax.experimental.pallas import tpu as pltpu
from jax.experimental.pallas import tpu_sc as plsc

NLEV = 4
NIMG = 8
N = NIMG * 512 * 512
NW = 32
ROWS_W = 512 // (NW // NIMG)
CHROWS = 64
NCHUNK = ROWS_W // CHROWS
B = 1024
EMAX = 8.0
INVW = B / EMAX
NROW = 8
PACK = 1024.0

_mesh = plsc.VectorSubcoreMesh(core_axis_name="c", subcore_axis_name="s")


@functools.partial(
    pl.kernel,
    mesh=_mesh,
    out_type=jax.ShapeDtypeStruct((NW, NROW, B), jnp.float32),
    scratch_types=[
        pltpu.VMEM((CHROWS, 512), jnp.int32),
        pltpu.VMEM((CHROWS, 512), jnp.float32),
        pltpu.VMEM((CHROWS, 512), jnp.float32),
        pltpu.VMEM((NROW * B,), jnp.float32),
        pltpu.SemaphoreType.DMA,
        pltpu.SemaphoreType.DMA,
    ],
    compiler_params=pltpu.CompilerParams(
        needs_layout_passes=False, use_tc_tiling_on_sc=True
    ),
)
def _sc_hist(lg_hbm, lab_hbm, out_hbm, lbuf, gbuf0, gbuf1, hist, sem0, sem1):
    w = lax.axis_index("s") * 2 + lax.axis_index("c")
    img = lax.div(w, 4)
    row0 = lax.rem(w, 4) * ROWS_W

    zeros = jnp.zeros((16,), jnp.float32)

    @pl.loop(0, NROW * B // 16)
    def _zero(i):
        hist[pl.ds(i * 16, 16)] = zeros

    magic = jnp.float32(2.0**23)

    gbufs = (gbuf0, gbuf1)
    sems = (sem0, sem1)
    units = [(c, lvl) for c in range(NCHUNK) for lvl in range(NLEV)]

    def _start(u, slot):
        c, lvl = units[u]
        r = row0 + c * CHROWS
        return pltpu.async_copy(
            lg_hbm.at[lvl, img, pl.ds(r, CHROWS), :], gbufs[slot], sems[slot]
        )

    handles = [None, None]
    handles[0] = _start(0, 0)
    for u, (c, lvl) in enumerate(units):
        slot = u % 2
        if lvl == 0:
            r = row0 + c * CHROWS
            pltpu.sync_copy(lab_hbm.at[img, pl.ds(r, CHROWS), :], lbuf)
        if u + 1 < len(units):
            handles[1 - slot] = _start(u + 1, 1 - slot)
        handles[slot].wait()
        gbuf = gbufs[slot]

        @plsc.parallel_loop(0, CHROWS * 512 // 16, unroll=16)
        def _body(v, lvl=lvl, gbuf=gbuf):
            rr = lax.shift_right_logical(v, 5)
            cc = lax.shift_left(jnp.bitwise_and(v, 31), 4)
            l = gbuf[rr, pl.ds(cc, 16)]
            y = lbuf[rr, pl.ds(cc, 16)]
            neg = jnp.bitwise_xor(
                plsc.bitcast(l, jnp.int32), lax.shift_left(y, 31)
            )
            e = 1.0 + plsc.bitcast(neg, jnp.float32)
            ec = jnp.minimum(e, jnp.float32(EMAX - 0.01))
            tb = plsc.bitcast(ec * INVW + magic, jnp.int32)
            bin_ = jnp.bitwise_and(tb, B - 1)
            idx = jnp.bitwise_or(
                jnp.bitwise_or(bin_, lax.shift_left(y, 12)), lvl * B
            )
            m = e > 0.0
            plsc.addupdate_scatter(hist, [idx], e + PACK, mask=m)

    for r in range(NROW):
        pltpu.sync_copy(hist.at[pl.ds(r * B, B)], out_hbm.at[w, r])


_BROWS = 256
_GRID_BCE = NIMG * (512 // _BROWS)


def _tc_bce_body(lg_ref, lab_ref, out_ref, acc):
    k = pl.program_id(0)

    @pl.when(k == 0)
    def _init():
        for i in range(NLEV + 1):
            acc[i] = 0.0

    y = lab_ref[0].astype(jnp.float32)
    acc[NLEV] += jnp.sum(y)
    for i in range(NLEV):
        l = lg_ref[i, 0]
        t = jnp.maximum(l, 0.0) - l * y + jnp.log1p(jnp.exp(-jnp.abs(l)))
        acc[i] += jnp.sum(t)

    @pl.when(k == _GRID_BCE - 1)
    def _emit():
        for i in range(NLEV + 1):
            out_ref[i] = acc[i]


_tc_bce = pl.pallas_call(
    _tc_bce_body,
    grid=(_GRID_BCE,),
    in_specs=[
        pl.BlockSpec(
            (NLEV, 1, _BROWS, 512), lambda k: (0, k // 2, k % 2, 0)
        ),
        pl.BlockSpec((1, _BROWS, 512), lambda k: (k // 2, k % 2, 0)),
    ],
    out_specs=pl.BlockSpec(memory_space=pltpu.SMEM),
    out_shape=jax.ShapeDtypeStruct((NLEV + 1,), jnp.float32),
    scratch_shapes=[pltpu.SMEM((NLEV + 1,), jnp.float32)],
    compiler_params=pltpu.CompilerParams(
        dimension_semantics=("arbitrary",),
    ),
)


def _tc_combine_body(hist_ref, bce_ref, out_ref):
    hp = hist_ref[...]
    c_all = jnp.floor(hp * (1.0 / PACK))
    cnt = jnp.sum(c_all, axis=0)
    ssum = jnp.sum(hp - PACK * c_all, axis=0)
    r = lax.broadcasted_iota(jnp.int32, (B, B), 0)
    c2 = lax.broadcasted_iota(jnp.int32, (B, B), 1)
    tri = (r > c2).astype(jnp.float32)
    s = jnp.dot(cnt, tri, preferred_element_type=jnp.float32)
    cn, cp = cnt[0:4], cnt[4:8]
    sn, sp = ssum[0:4], ssum[4:8]
    n_eff = s[0:4] + 0.5 * cn
    p_eff = s[4:8] + 0.5 * cp
    pf = bce_ref[NLEV]
    p2 = N - pf
    den1 = pf + n_eff
    dot1 = jnp.sum(sp / den1 + sn * (pf - p_eff) / (den1 * (den1 + 1.0)))
    den2 = p2 + p_eff
    dot2 = jnp.sum(sn / den2 + sp * (p2 - n_eff) / (den2 * (den2 + 1.0)))
    lov_total = 0.5 * (dot1 + dot2)
    bce_total = bce_ref[0] + bce_ref[1] + bce_ref[2] + bce_ref[3]
    out_ref[0, 0] = 0.1 * (bce_total / N + lov_total)


_tc_combine = pl.pallas_call(
    _tc_combine_body,
    in_specs=[
        pl.BlockSpec((NW, NROW, B), lambda: (0, 0, 0)),
        pl.BlockSpec(memory_space=pltpu.SMEM),
    ],
    out_specs=pl.BlockSpec(memory_space=pltpu.SMEM),
    out_shape=jax.ShapeDtypeStruct((1, 1), jnp.float32),
)


def kernel(logits_deep, label):
    hist = _sc_hist(logits_deep, label)
    bce = _tc_bce(logits_deep, label)
    out = _tc_combine(hist, bce)
    return out[0, 0]


# speedup vs baseline: 1.1057x; 1.1057x over previous
import functools

import jax
import jax.numpy as jnp
from jax import lax
from jax.experimental import pallas as pl
from jax.experimental.pallas import tpu as pltpu
from jax.experimental.pallas import tpu_sc as plsc

NLEV = 4
NIMG = 8
N = NIMG * 512 * 512
NW = 32
ROWS_W = 512 // (NW // NIMG)
CHROWS = 64
NCHUNK = ROWS_W // CHROWS
B = 1024
EMAX = 8.0
INVW = B / EMAX
NROW = 8
PACK = 1024.0

_mesh = plsc.VectorSubcoreMesh(core_axis_name="c", subcore_axis_name="s")


@functools.partial(
    pl.kernel,
    mesh=_mesh,
    out_type=jax.ShapeDtypeStruct((NW, NROW, B), jnp.float32),
    scratch_types=[
        pltpu.VMEM((CHROWS, 512), jnp.int32),
        pltpu.VMEM((CHROWS, 512), jnp.float32),
        pltpu.VMEM((CHROWS, 512), jnp.float32),
        pltpu.VMEM((NROW * B,), jnp.float32),
        pltpu.SemaphoreType.DMA,
        pltpu.SemaphoreType.DMA,
    ],
    compiler_params=pltpu.CompilerParams(
        needs_layout_passes=False, use_tc_tiling_on_sc=True
    ),
)
def _sc_hist(lg_hbm, lab_hbm, out_hbm, lbuf, gbuf0, gbuf1, hist, sem0, sem1):
    w = lax.axis_index("s") * 2 + lax.axis_index("c")
    img = lax.div(w, 4)
    row0 = lax.rem(w, 4) * ROWS_W

    zeros = jnp.zeros((16,), jnp.float32)

    @pl.loop(0, NROW * B // 16)
    def _zero(i):
        hist[pl.ds(i * 16, 16)] = zeros

    magic = jnp.float32(2.0**23)

    gbufs = (gbuf0, gbuf1)
    sems = (sem0, sem1)
    units = [(c, lvl) for c in range(NCHUNK) for lvl in range(NLEV)]

    def _start(u, slot):
        c, lvl = units[u]
        r = row0 + c * CHROWS
        return pltpu.async_copy(
            lg_hbm.at[lvl, img, pl.ds(r, CHROWS), :], gbufs[slot], sems[slot]
        )

    handles = [None, None]
    handles[0] = _start(0, 0)
    for u, (c, lvl) in enumerate(units):
        slot = u % 2
        if lvl == 0:
            r = row0 + c * CHROWS
            pltpu.sync_copy(lab_hbm.at[img, pl.ds(r, CHROWS), :], lbuf)
        if u + 1 < len(units):
            handles[1 - slot] = _start(u + 1, 1 - slot)
        handles[slot].wait()
        gbuf = gbufs[slot]

        @plsc.parallel_loop(0, CHROWS * 512 // 16, unroll=8)
        def _body(v, lvl=lvl, gbuf=gbuf):
            rr = lax.shift_right_logical(v, 5)
            cc = lax.shift_left(jnp.bitwise_and(v, 31), 4)
            l = gbuf[rr, pl.ds(cc, 16)]
            y = lbuf[rr, pl.ds(cc, 16)]
            neg = jnp.bitwise_xor(
                plsc.bitcast(l, jnp.int32), lax.shift_left(y, 31)
            )
            e = 1.0 + plsc.bitcast(neg, jnp.float32)
            ec = jnp.minimum(e, jnp.float32(EMAX - 0.01))
            tb = plsc.bitcast(ec * INVW + magic, jnp.int32)
            bin_ = jnp.bitwise_and(tb, B - 1)
            idx = jnp.bitwise_or(
                jnp.bitwise_or(bin_, lax.shift_left(y, 12)), lvl * B
            )
            m = e > 0.0
            plsc.addupdate_scatter(hist, [idx], e + PACK, mask=m)

    for r in range(NROW):
        pltpu.sync_copy(hist.at[pl.ds(r * B, B)], out_hbm.at[w, r])


_BROWS = 256
_GRID_BCE = NIMG * (512 // _BROWS)


def _tc_bce_body(lg_ref, lab_ref, out_ref, acc):
    k = pl.program_id(0)

    @pl.when(k == 0)
    def _init():
        for i in range(NLEV + 1):
            acc[i] = 0.0

    y = lab_ref[0].astype(jnp.float32)
    acc[NLEV] += jnp.sum(y)
    for i in range(NLEV):
        l = lg_ref[i, 0]
        t = jnp.maximum(l, 0.0) - l * y + jnp.log1p(jnp.exp(-jnp.abs(l)))
        acc[i] += jnp.sum(t)

    @pl.when(k == _GRID_BCE - 1)
    def _emit():
        for i in range(NLEV + 1):
            out_ref[i] = acc[i]


_tc_bce = pl.pallas_call(
    _tc_bce_body,
    grid=(_GRID_BCE,),
    in_specs=[
        pl.BlockSpec(
            (NLEV, 1, _BROWS, 512), lambda k: (0, k // 2, k % 2, 0)
        ),
        pl.BlockSpec((1, _BROWS, 512), lambda k: (k // 2, k % 2, 0)),
    ],
    out_specs=pl.BlockSpec(memory_space=pltpu.SMEM),
    out_shape=jax.ShapeDtypeStruct((NLEV + 1,), jnp.float32),
    scratch_shapes=[pltpu.SMEM((NLEV + 1,), jnp.float32)],
    compiler_params=pltpu.CompilerParams(
        dimension_semantics=("arbitrary",),
    ),
)


def _tc_combine_body(hist_ref, bce_ref, out_ref):
    hp = hist_ref[...]
    c_all = jnp.floor(hp * (1.0 / PACK))
    cnt = jnp.sum(c_all, axis=0)
    ssum = jnp.sum(hp - PACK * c_all, axis=0)
    r = lax.broadcasted_iota(jnp.int32, (B, B), 0)
    c2 = lax.broadcasted_iota(jnp.int32, (B, B), 1)
    tri = (r > c2).astype(jnp.float32)
    s = jnp.dot(cnt, tri, preferred_element_type=jnp.float32)
    cn, cp = cnt[0:4], cnt[4:8]
    sn, sp = ssum[0:4], ssum[4:8]
    n_eff = s[0:4] + 0.5 * cn
    p_eff = s[4:8] + 0.5 * cp
    pf = bce_ref[NLEV]
    p2 = N - pf
    den1 = pf + n_eff
    dot1 = jnp.sum(sp / den1 + sn * (pf - p_eff) / (den1 * (den1 + 1.0)))
    den2 = p2 + p_eff
    dot2 = jnp.sum(sn / den2 + sp * (p2 - n_eff) / (den2 * (den2 + 1.0)))
    lov_total = 0.5 * (dot1 + dot2)
    bce_total = bce_ref[0] + bce_ref[1] + bce_ref[2] + bce_ref[3]
    out_ref[0, 0] = 0.1 * (bce_total / N + lov_total)


_tc_combine = pl.pallas_call(
    _tc_combine_body,
    in_specs=[
        pl.BlockSpec((NW, NROW, B), lambda: (0, 0, 0)),
        pl.BlockSpec(memory_space=pltpu.SMEM),
    ],
    out_specs=pl.BlockSpec(memory_space=pltpu.SMEM),
    out_shape=jax.ShapeDtypeStruct((1, 1), jnp.float32),
)


def kernel(logits_deep, label):
    hist = _sc_hist(logits_deep, label)
    bce = _tc_bce(logits_deep, label)
    out = _tc_combine(hist, bce)
    return out[0, 0]
